# baseline (speedup 1.0000x reference)
import functools

import jax
import jax.numpy as jnp
from jax import lax
from jax.experimental import pallas as pl
from jax.experimental.pallas import tpu as pltpu
from jax.experimental.pallas import tpu_sc as plsc

VOCAB = 1000000
EMB = 16
NIG = 5
NGG = 21
OUT = 10
BATCH = 16384

NC, NS, L = 2, 16, 16
NW = NC * NS
BPW = BATCH // NW
CH = 128
NCHUNK = BPW // CH
NG = BPW // L

WCOLS = EMB + NIG + NGG
WFLAT = OUT * (WCOLS + 1)


def _body(idx_hbm, ig_hbm, gg_hbm, table_hbm, w_hbm, wsplat_hbm, out_hbm,
          idx_v, ig_v, gg_v, rows_v, w_v, wsplat_v, out_v, sem):
    wid = lax.axis_index("s") * NC + lax.axis_index("c")

    pltpu.sync_copy(idx_hbm.at[wid], idx_v)
    pltpu.sync_copy(ig_hbm.at[wid], ig_v)
    pltpu.sync_copy(gg_hbm.at[wid], gg_v)
    pltpu.sync_copy(w_hbm, w_v)
    pltpu.sync_copy(wsplat_hbm, wsplat_v)

    copies = [
        pltpu.async_copy(table_hbm.at[idx_v.at[c]],
                         rows_v.at[pl.ds(c * CH, CH)], sem)
        for c in range(NCHUNK)
    ]

    iota = lax.iota(jnp.int32, L)

    for cp in copies:
        cp.wait()

    def g_body(g, carry):
        gbase = g * L
        row_ids = gbase + iota
        es = [
            jnp.maximum(
                plsc.load_gather(rows_v,
                                 [row_ids, jnp.full((L,), k, jnp.int32)]),
                0.0)
            for k in range(EMB)
        ]
        igv = ig_v[pl.ds(gbase, L)]
        ggv = gg_v[pl.ds(gbase, L)]
        for j in range(OUT):
            base = j * (WCOLS + 1)
            acc = wsplat_v[j, EMB]
            for k in range(EMB):
                acc = acc + es[k] * wsplat_v[j, k]
            acc = acc + plsc.load_gather(w_v, [(base + EMB) + igv])
            acc = acc + plsc.load_gather(w_v, [(base + EMB + NIG) + ggv])
            plsc.store_scatter(out_v, [row_ids, jnp.full((L,), j, jnp.int32)],
                               acc)
        return carry

    lax.fori_loop(0, NG, g_body, 0)

    pltpu.sync_copy(out_v, out_hbm.at[pl.ds(wid * BPW, BPW)])


@jax.jit
def _run(idx3, ig2, gg2, emb_table, wflat, wsplat):
    mesh = plsc.VectorSubcoreMesh(core_axis_name="c", subcore_axis_name="s",
                                  num_cores=NC, num_subcores=NS)
    return pl.kernel(
        _body,
        out_type=jax.ShapeDtypeStruct((BATCH, OUT), jnp.float32),
        mesh=mesh,
        compiler_params=pltpu.CompilerParams(
            needs_layout_passes=False, use_tc_tiling_on_sc=False),
        scratch_types=[
            pltpu.VMEM((NCHUNK, CH), jnp.int32),
            pltpu.VMEM((BPW,), jnp.int32),
            pltpu.VMEM((BPW,), jnp.int32),
            pltpu.VMEM((BPW, EMB), jnp.float32),
            pltpu.VMEM((WFLAT + 2,), jnp.float32),
            pltpu.VMEM((OUT, EMB + 1, L), jnp.float32),
            pltpu.VMEM((BPW, OUT), jnp.float32),
            pltpu.SemaphoreType.DMA,
        ],
    )(idx3, ig2, gg2, emb_table, wflat, wsplat)


def kernel(item_indices, index_group_indices, garment_group_indices,
           emb_table, W, b):
    idx3 = item_indices.astype(jnp.int32).reshape(NW, NCHUNK, CH)
    ig2 = index_group_indices.astype(jnp.int32).reshape(NW, BPW)
    gg2 = garment_group_indices.astype(jnp.int32).reshape(NW, BPW)
    wb = jnp.concatenate([W, b[:, None]], axis=1)
    wflat = jnp.pad(wb.reshape(WFLAT), (0, 2))
    wsplat = jnp.broadcast_to(
        jnp.concatenate([W[:, :EMB], b[:, None]], axis=1)[:, :, None],
        (OUT, EMB + 1, L)).astype(jnp.float32)
    return _run(idx3, ig2, gg2, emb_table, wflat, wsplat)
